# baseline (speedup 1.0000x reference)
import jax
import jax.numpy as jnp
from jax import lax
from jax.experimental import pallas as pl
from jax.experimental.pallas import tpu as pltpu
from jax.experimental.pallas import tpu_sc as plsc

NC, NS, L = 2, 16, 16
NW = NC * NS
ROWS_PER_SUB = 8192 // NS
X_PER_W = 1048576 // NW


def _sc_body(book_hbm, x_hbm, zq_hbm, idx_hbm, part_hbm,
             book_v, xbuf, rowbuf, idxbuf, bk_v, lv,
             stage_f, stage_i, lstage, shared_v, sem_in, sem_zq):
    c = lax.axis_index("c")
    s = lax.axis_index("s")
    wid = c * NS + s
    lane = lax.iota(jnp.int32, L)
    zero16f = jnp.zeros((L,), jnp.float32)
    zero16i = jnp.zeros((L,), jnp.int32)

    in_desc = pltpu.async_copy(
        x_hbm.at[pl.ds(pl.multiple_of(wid * X_PER_W, X_PER_W), X_PER_W)],
        xbuf, sem_in)

    pltpu.sync_copy(
        book_hbm.at[pl.ds(pl.multiple_of(s * ROWS_PER_SUB * 64, 32768),
                          ROWS_PER_SUB * 64)],
        book_v)

    def chunk_body(c16, carry):
        mv, mr = carry
        base = c16 * (16 * 64)
        acc = zero16f
        for col in range(64):
            v = plsc.load_gather(book_v, [base + col + lane * 64])
            acc = acc + v * v
        m = jnp.min(acc)
        ffs = plsc.all_reduce_ffs(acc == m)
        row = s * ROWS_PER_SUB + c16 * 16 + ffs
        mvec = zero16f + m
        better = mvec < mv
        return jnp.where(better, mvec, mv), jnp.where(better, row, mr)

    big = zero16f + jnp.float32(3.4e38)
    mv, mr = lax.fori_loop(0, ROWS_PER_SUB // 16, chunk_body, (big, zero16i))

    stage_f[...] = mv
    stage_i[...] = plsc.bitcast(mr, jnp.float32)
    pltpu.sync_copy(stage_f, shared_v.at[s])
    pltpu.sync_copy(stage_i, shared_v.at[NS + s])
    plsc.subcore_barrier()
    pltpu.sync_copy(shared_v, lv)
    vals = plsc.load_gather(lv, [lane, zero16i])
    m = jnp.min(vals)
    swin = plsc.all_reduce_ffs(vals == m)
    jbits = plsc.load_gather(lv, [swin + NS, zero16i])
    jvec = plsc.bitcast(jbits, jnp.int32)
    jsc = jnp.max(jvec)
    pltpu.sync_copy(
        book_hbm.at[pl.ds(pl.multiple_of(jsc * 64, 64), 64)], bk_v)

    def idx_body(i, _):
        idxbuf[pl.ds(i * L, L)] = jvec
        return 0
    lax.fori_loop(0, 512 // L, idx_body, 0)
    pltpu.sync_copy(
        idxbuf, idx_hbm.at[pl.ds(pl.multiple_of(wid * 512, 512), 512)])

    vw = plsc.load_gather(bk_v, [(lane % 2) * 32 + wid])
    def row_body(i, _):
        rowbuf[pl.ds(i * L, L)] = vw
        return 0
    lax.fori_loop(0, 2048 // L, row_body, 0)
    zq_descs = []
    for b in range(16):
        off = pl.multiple_of((b * 32 + wid) * 2048, 2048)
        zq_descs.append(
            pltpu.async_copy(rowbuf, zq_hbm.at[pl.ds(off, 2048)], sem_zq))

    in_desc.wait()
    p = [bk_v[pl.ds(k * L, L)] for k in range(4)]
    def g_body(g, acc):
        base = g * 64
        for k in range(4):
            v = xbuf[pl.ds(base + k * L, L)]
            d = v - p[k]
            acc = acc + d * d
        return acc
    acc = lax.fori_loop(0, X_PER_W // 64, g_body, zero16f)
    lstage[...] = acc
    pltpu.sync_copy(lstage, part_hbm.at[wid])

    for dsc in zq_descs:
        dsc.wait()


def _part_body(part_ref, loss_ref):
    total = jnp.sum(part_ref[...])
    loss = jnp.float32(1.25) * total / jnp.float32(1048576.0)
    loss_ref[...] = loss.reshape(1, 1)


def kernel(input, book):
    x = input.reshape(1048576)
    bookf = book.reshape(524288)
    mesh = plsc.VectorSubcoreMesh(
        core_axis_name="c", subcore_axis_name="s",
        num_cores=NC, num_subcores=NS)
    sc = pl.kernel(
        _sc_body,
        out_type=[
            jax.ShapeDtypeStruct((1048576,), jnp.float32),
            jax.ShapeDtypeStruct((16384,), jnp.int32),
            jax.ShapeDtypeStruct((NW, L), jnp.float32),
        ],
        mesh=mesh,
        compiler_params=pltpu.CompilerParams(needs_layout_passes=False),
        scratch_types=[
            pltpu.VMEM((ROWS_PER_SUB * 64,), jnp.float32),
            pltpu.VMEM((X_PER_W,), jnp.float32),
            pltpu.VMEM((2048,), jnp.float32),
            pltpu.VMEM((512,), jnp.int32),
            pltpu.VMEM((64,), jnp.float32),
            pltpu.VMEM((2 * NS, L), jnp.float32),
            pltpu.VMEM((L,), jnp.float32),
            pltpu.VMEM((L,), jnp.float32),
            pltpu.VMEM((L,), jnp.float32),
            pltpu.VMEM_SHARED((2 * NS, L), jnp.float32),
            pltpu.SemaphoreType.DMA,
            pltpu.SemaphoreType.DMA,
        ],
    )
    zq, idx, part = sc(bookf, x)
    lossm = pl.pallas_call(
        _part_body,
        out_shape=jax.ShapeDtypeStruct((1, 1), jnp.float32),
    )(part)
    return (zq.reshape(16, 32, 64, 32), idx, lossm.reshape(()))
